# baseline (speedup 1.0000x reference)
import functools

import jax
import jax.numpy as jnp
from jax import lax
from jax.experimental import pallas as pl
from jax.experimental.pallas import tpu as pltpu
from jax.experimental.pallas import tpu_sc as plsc

N_NODES = 10000
N_PAD = 10240
E_EDGES = 320000
F = 128
K_STEPS = 10
ALPHA = 0.1

NC = 2
NS = 16
CHUNK = 128
CPW = 80
SCH = 64
NBUF = 4
E_PAD = NC * NS * CPW * CHUNK
ROWS_PER_TILE = N_PAD // NS
PACK_SHIFT = 14


def _mesh():
    return plsc.VectorSubcoreMesh(core_axis_name="c", subcore_axis_name="s")


@functools.partial(
    pl.kernel,
    out_type=jax.ShapeDtypeStruct((NC, N_PAD), jnp.float32),
    mesh=_mesh(),
    compiler_params=pltpu.CompilerParams(needs_layout_passes=False),
    scratch_types=[
        pltpu.VMEM((CPW, CHUNK), jnp.int32),
        pltpu.VMEM((N_PAD,), jnp.float32),
        pltpu.VMEM_SHARED((NS, N_PAD), jnp.float32),
        pltpu.VMEM((ROWS_PER_TILE,), jnp.float32),
        pltpu.VMEM((ROWS_PER_TILE,), jnp.float32),
    ],
)
def _degree_kernel(pk_hbm, deg_out, pkv, hist, shared, acc, tmp):
    c = lax.axis_index("c")
    s = lax.axis_index("s")
    pltpu.sync_copy(pk_hbm.at[c, s], pkv)

    zeros16 = jnp.zeros((16,), jnp.float32)
    ones16 = jnp.ones((16,), jnp.float32)
    mask = jnp.full((16,), (1 << PACK_SHIFT) - 1, jnp.int32)

    def zero_body(i, carry):
        hist[pl.ds(i * 16, 16)] = zeros16
        return carry
    lax.fori_loop(0, N_PAD // 16, zero_body, 0)

    def hist_body(i, carry):
        j = i // (CHUNK // 16)
        k = i % (CHUNK // 16)
        idx = lax.bitwise_and(pkv[j, pl.ds(k * 16, 16)], mask)
        plsc.addupdate_scatter(hist, [idx], ones16)
        return carry
    lax.fori_loop(0, CPW * (CHUNK // 16), hist_body, 0)

    pltpu.sync_copy(hist, shared.at[s])
    plsc.subcore_barrier()

    base = s * ROWS_PER_TILE

    def z2(i, carry):
        acc[pl.ds(i * 16, 16)] = zeros16
        return carry
    lax.fori_loop(0, ROWS_PER_TILE // 16, z2, 0)

    for t in range(NS):
        pltpu.sync_copy(shared.at[t, pl.ds(base, ROWS_PER_TILE)], tmp)

        def add_body(i, carry):
            sl = pl.ds(i * 16, 16)
            acc[sl] = acc[sl] + tmp[sl]
            return carry
        lax.fori_loop(0, ROWS_PER_TILE // 16, add_body, 0)

    pltpu.sync_copy(acc, deg_out.at[c, pl.ds(base, ROWS_PER_TILE)])


@functools.partial(
    pl.kernel,
    out_type=jax.ShapeDtypeStruct((NC, N_PAD, F), jnp.float32),
    mesh=_mesh(),
    compiler_params=pltpu.CompilerParams(needs_layout_passes=False),
    scratch_types=[
        pltpu.VMEM((CPW, CHUNK), jnp.int32),
        pltpu.VMEM((SCH,), jnp.int32),
        pltpu.VMEM((SCH,), jnp.int32),
        pltpu.VMEM((SCH,), jnp.int32),
        pltpu.VMEM((SCH,), jnp.int32),
        pltpu.VMEM((SCH,), jnp.int32),
        pltpu.VMEM((SCH,), jnp.int32),
        pltpu.VMEM((SCH,), jnp.int32),
        pltpu.VMEM((SCH,), jnp.int32),
        pltpu.VMEM((NBUF * SCH, F), jnp.float32),
        pltpu.VMEM_SHARED((N_PAD, F), jnp.float32),
        pltpu.SemaphoreType.DMA,
        pltpu.SemaphoreType.DMA,
    ],
)
def _scatter_kernel(g_hbm, pk_hbm, aggs_out,
                    pkv, si0, si1, si2, si3, di0, di1, di2, di3,
                    rows, agg_sh, sem_g, sem_s):
    c = lax.axis_index("c")
    s = lax.axis_index("s")
    idx_cp = pltpu.async_copy(pk_hbm.at[c, s], pkv, sem_s)
    sis = (si0, si1, si2, si3)
    dis = (di0, di1, di2, di3)

    zeros16 = jnp.zeros((16,), jnp.float32)
    mask = jnp.full((16,), (1 << PACK_SHIFT) - 1, jnp.int32)

    def zrow(i, carry):
        r = i // (F // 16)
        k = i % (F // 16)
        rows[r, pl.ds(k * 16, 16)] = zeros16
        return carry
    lax.fori_loop(0, (NBUF * SCH) * (F // 16), zrow, 0)

    base = s * ROWS_PER_TILE
    _sz = NBUF * SCH
    _sizes = [_sz] * (ROWS_PER_TILE // _sz)
    if ROWS_PER_TILE % _sz:
        _sizes.append(ROWS_PER_TILE % _sz)
    off = 0
    for z in _sizes:
        pltpu.async_copy(rows.at[pl.ds(0, z)],
                         agg_sh.at[pl.ds(base + off, z)], sem_g)
        off += z
    off = 0
    for z in _sizes:
        pltpu.make_async_copy(rows.at[pl.ds(0, z)],
                              agg_sh.at[pl.ds(base + off, z)], sem_g).wait()
        off += z
    idx_cp.wait()
    plsc.subcore_barrier()

    spg = CHUNK // SCH
    ngrp = CPW * spg // NBUF

    def grp(j, carry):
        @pl.when(j > 0)
        def _():
            for b in range(NBUF):
                pltpu.make_async_copy(
                    rows.at[pl.ds(b * SCH, SCH)],
                    agg_sh.at[dis[b]], sem_s).wait()
        for b in range(NBUF):
            for v in range(SCH // 16):
                e = (j * NBUF + b) * SCH + v * 16
                p = pkv[e // CHUNK, pl.ds(e % CHUNK, 16)]
                sl = pl.ds(v * 16, 16)
                sis[b][sl] = lax.shift_right_logical(p, PACK_SHIFT)
                dis[b][sl] = lax.bitwise_and(p, mask)
        for b in range(NBUF):
            pltpu.async_copy(
                g_hbm.at[sis[b]], rows.at[pl.ds(b * SCH, SCH)], sem_g)
        for b in range(NBUF):
            pltpu.make_async_copy(
                g_hbm.at[sis[b]], rows.at[pl.ds(b * SCH, SCH)], sem_g).wait()
        for b in range(NBUF):
            pltpu.async_copy(
                rows.at[pl.ds(b * SCH, SCH)], agg_sh.at[dis[b]], sem_s,
                add=True)
        return carry

    lax.fori_loop(0, ngrp, grp, 0)
    for b in range(NBUF):
        pltpu.make_async_copy(
            rows.at[pl.ds(b * SCH, SCH)],
            agg_sh.at[dis[b]], sem_s).wait()
    plsc.subcore_barrier()

    off = 0
    for z in _sizes:
        pltpu.sync_copy(agg_sh.at[pl.ds(base + off, z)],
                        rows.at[pl.ds(0, z)])
        pltpu.sync_copy(rows.at[pl.ds(0, z)],
                        aggs_out.at[c].at[pl.ds(base + off, z)])
        off += z


_BR = 2048


def _tc1_body(x_ref, w_ref, b_ref, degp_ref, h0_ref, g_ref, dinv_ref):
    deg = degp_ref[0, :] + degp_ref[1, :] + 1.0
    dinv = jnp.where(deg > 0, lax.rsqrt(deg), 0.0)
    h0 = jnp.dot(x_ref[...], w_ref[...], preferred_element_type=jnp.float32)
    h0 = h0 + b_ref[...]
    h0_ref[...] = h0
    g_ref[...] = h0 * dinv[:, None]
    dinv_ref[...] = dinv[:, None]


def _tc1(x, W1, b1, degp):
    grid = (N_PAD // _BR,)
    return pl.pallas_call(
        _tc1_body,
        grid=grid,
        in_specs=[
            pl.BlockSpec((_BR, F), lambda i: (i, 0)),
            pl.BlockSpec((F, F), lambda i: (0, 0)),
            pl.BlockSpec((1, F), lambda i: (0, 0)),
            pl.BlockSpec((NC, _BR), lambda i: (0, i)),
        ],
        out_specs=[
            pl.BlockSpec((_BR, F), lambda i: (i, 0)),
            pl.BlockSpec((_BR, F), lambda i: (i, 0)),
            pl.BlockSpec((_BR, 1), lambda i: (i, 0)),
        ],
        out_shape=[
            jax.ShapeDtypeStruct((N_PAD, F), jnp.float32),
            jax.ShapeDtypeStruct((N_PAD, F), jnp.float32),
            jax.ShapeDtypeStruct((N_PAD, 1), jnp.float32),
        ],
    )(x, W1, b1.reshape(1, F), degp)


def _combine_mid_body(a0_ref, a1_ref, g_ref, h0_ref, dinv_ref, gn_ref):
    dinv = dinv_ref[...]
    raw = a0_ref[...] + a1_ref[...] + g_ref[...]
    h = (1.0 - ALPHA) * dinv * raw + ALPHA * h0_ref[...]
    gn_ref[...] = h * dinv


def _combine_mid(aggs, g, h0, dinv):
    grid = (N_PAD // _BR,)
    bs = pl.BlockSpec((_BR, F), lambda i: (i, 0))
    return pl.pallas_call(
        _combine_mid_body,
        grid=grid,
        in_specs=[bs, bs, bs, bs, pl.BlockSpec((_BR, 1), lambda i: (i, 0))],
        out_specs=bs,
        out_shape=jax.ShapeDtypeStruct((N_PAD, F), jnp.float32),
    )(aggs[0], aggs[1], g, h0, dinv)


def _combine_lin_body(a0_ref, a1_ref, g_ref, h0_ref, dinv_ref, w_ref, b_ref,
                      h0n_ref, gn_ref):
    dinv = dinv_ref[...]
    raw = a0_ref[...] + a1_ref[...] + g_ref[...]
    h = (1.0 - ALPHA) * dinv * raw + ALPHA * h0_ref[...]
    h = jnp.maximum(h, 0.0)
    h2 = jnp.dot(h, w_ref[...], preferred_element_type=jnp.float32)
    h2 = h2 + b_ref[...]
    h0n_ref[...] = h2
    gn_ref[...] = h2 * dinv


def _combine_lin(aggs, g, h0, dinv, W2, b2):
    grid = (N_PAD // _BR,)
    bs = pl.BlockSpec((_BR, F), lambda i: (i, 0))
    return pl.pallas_call(
        _combine_lin_body,
        grid=grid,
        in_specs=[bs, bs, bs, bs,
                  pl.BlockSpec((_BR, 1), lambda i: (i, 0)),
                  pl.BlockSpec((F, F), lambda i: (0, 0)),
                  pl.BlockSpec((1, F), lambda i: (0, 0))],
        out_specs=[bs, bs],
        out_shape=[
            jax.ShapeDtypeStruct((N_PAD, F), jnp.float32),
            jax.ShapeDtypeStruct((N_PAD, F), jnp.float32),
        ],
    )(aggs[0], aggs[1], g, h0, dinv, W2, b2.reshape(1, F))


def _combine_final_body(a0_ref, a1_ref, g_ref, h0_ref, dinv_ref, out_ref):
    dinv = dinv_ref[...]
    raw = a0_ref[...] + a1_ref[...] + g_ref[...]
    out_ref[...] = (1.0 - ALPHA) * dinv * raw + ALPHA * h0_ref[...]


def _combine_final(aggs, g, h0, dinv):
    grid = (N_PAD // _BR,)
    bs = pl.BlockSpec((_BR, F), lambda i: (i, 0))
    return pl.pallas_call(
        _combine_final_body,
        grid=grid,
        in_specs=[bs, bs, bs, bs, pl.BlockSpec((_BR, 1), lambda i: (i, 0))],
        out_specs=bs,
        out_shape=jax.ShapeDtypeStruct((N_PAD, F), jnp.float32),
    )(aggs[0], aggs[1], g, h0, dinv)


def kernel(x, edge_index, W1, b1, W2, b2):
    x_pad = jnp.zeros((N_PAD, F), jnp.float32).at[:N_NODES].set(x)
    packed = jnp.left_shift(edge_index[0], PACK_SHIFT) | edge_index[1]
    pad_val = jnp.int32((N_NODES << PACK_SHIFT) | N_NODES)
    pad = jnp.full((E_PAD - E_EDGES,), pad_val, jnp.int32)
    pk = jnp.concatenate([packed, pad]).reshape(NC, NS, CPW, CHUNK)

    degp = _degree_kernel(pk)
    h0, g, dinv = _tc1(x_pad, W1, b1, degp)

    out = None
    for k in range(K_STEPS):
        aggs = _scatter_kernel(g, pk)
        if k < K_STEPS - 1:
            g = _combine_mid(aggs, g, h0, dinv)
        else:
            h0, g = _combine_lin(aggs, g, h0, dinv, W2, b2)

    for k in range(K_STEPS):
        aggs = _scatter_kernel(g, pk)
        if k < K_STEPS - 1:
            g = _combine_mid(aggs, g, h0, dinv)
        else:
            out = _combine_final(aggs, g, h0, dinv)

    return out[:N_NODES]


# speedup vs baseline: 1.4360x; 1.4360x over previous
import functools

import jax
import jax.numpy as jnp
from jax import lax
from jax.experimental import pallas as pl
from jax.experimental.pallas import tpu as pltpu
from jax.experimental.pallas import tpu_sc as plsc

N_NODES = 10000
N_PAD = 10240
E_EDGES = 320000
F = 128
K_STEPS = 10
ALPHA = 0.1

NC = 2
NS = 16
CHUNK = 128
CPW = 79
E_PAD = NC * NS * CPW * CHUNK
ROWS_PER_TILE = N_PAD // NS


def _mesh():
    return plsc.VectorSubcoreMesh(core_axis_name="c", subcore_axis_name="s")


@functools.partial(
    pl.kernel,
    out_type=jax.ShapeDtypeStruct((NC, N_PAD), jnp.float32),
    mesh=_mesh(),
    compiler_params=pltpu.CompilerParams(needs_layout_passes=False),
    scratch_types=[
        pltpu.VMEM((CPW, CHUNK), jnp.int32),
        pltpu.VMEM((N_PAD,), jnp.float32),
        pltpu.VMEM_SHARED((NS, N_PAD), jnp.float32),
        pltpu.VMEM((ROWS_PER_TILE,), jnp.float32),
        pltpu.VMEM((ROWS_PER_TILE,), jnp.float32),
    ],
)
def _degree_kernel(dst_hbm, deg_out, dstv, hist, shared, acc, tmp):
    c = lax.axis_index("c")
    s = lax.axis_index("s")
    pltpu.sync_copy(dst_hbm.at[c, s], dstv)

    zeros16 = jnp.zeros((16,), jnp.float32)
    ones16 = jnp.ones((16,), jnp.float32)

    def zero_body(i, carry):
        hist[pl.ds(i * 16, 16)] = zeros16
        return carry
    lax.fori_loop(0, N_PAD // 16, zero_body, 0)

    def hist_body(i, carry):
        j = i // (CHUNK // 16)
        k = i % (CHUNK // 16)
        idx = dstv[j, pl.ds(k * 16, 16)]
        plsc.addupdate_scatter(hist, [idx], ones16)
        return carry
    lax.fori_loop(0, CPW * (CHUNK // 16), hist_body, 0)

    pltpu.sync_copy(hist, shared.at[s])
    plsc.subcore_barrier()

    base = s * ROWS_PER_TILE

    def z2(i, carry):
        acc[pl.ds(i * 16, 16)] = zeros16
        return carry
    lax.fori_loop(0, ROWS_PER_TILE // 16, z2, 0)

    for t in range(NS):
        pltpu.sync_copy(shared.at[t, pl.ds(base, ROWS_PER_TILE)], tmp)

        def add_body(i, carry):
            sl = pl.ds(i * 16, 16)
            acc[sl] = acc[sl] + tmp[sl]
            return carry
        lax.fori_loop(0, ROWS_PER_TILE // 16, add_body, 0)

    pltpu.sync_copy(acc, deg_out.at[c, pl.ds(base, ROWS_PER_TILE)])


@functools.partial(
    pl.kernel,
    out_type=jax.ShapeDtypeStruct((NC, N_PAD, F), jnp.float32),
    mesh=_mesh(),
    compiler_params=pltpu.CompilerParams(needs_layout_passes=False),
    scratch_types=[
        pltpu.VMEM((CPW, CHUNK), jnp.int32),
        pltpu.VMEM((CPW, CHUNK), jnp.int32),
        pltpu.VMEM((CHUNK, F), jnp.float32),
        pltpu.VMEM_SHARED((N_PAD, F), jnp.float32),
    ],
)
def _scatter_kernel(g_hbm, src_hbm, dst_hbm, aggs_out,
                    srcv, dstv, rows, agg_sh):
    c = lax.axis_index("c")
    s = lax.axis_index("s")
    pltpu.sync_copy(src_hbm.at[c, s], srcv)
    pltpu.sync_copy(dst_hbm.at[c, s], dstv)

    zeros16 = jnp.zeros((16,), jnp.float32)

    def zrow(i, carry):
        r = i // (F // 16)
        k = i % (F // 16)
        rows[r, pl.ds(k * 16, 16)] = zeros16
        return carry
    lax.fori_loop(0, CHUNK * (F // 16), zrow, 0)

    base = s * ROWS_PER_TILE
    for q in range(ROWS_PER_TILE // CHUNK):
        pltpu.sync_copy(rows, agg_sh.at[pl.ds(base + q * CHUNK, CHUNK)])
    plsc.subcore_barrier()

    def step(j, carry):
        pltpu.sync_copy(g_hbm.at[srcv.at[j]], rows)
        pltpu.sync_copy(rows, agg_sh.at[dstv.at[j]], add=True)
        return carry

    lax.fori_loop(0, CPW, step, 0)
    plsc.subcore_barrier()

    sl = pl.ds(base, ROWS_PER_TILE)
    pltpu.sync_copy(agg_sh.at[sl], aggs_out.at[c].at[sl])


_BR = 2048


def _tc1_body(x_ref, w_ref, b_ref, degp_ref, h0_ref, g_ref, dinv_ref):
    deg = degp_ref[0, :] + degp_ref[1, :] + 1.0
    dinv = jnp.where(deg > 0, lax.rsqrt(deg), 0.0)
    h0 = jnp.dot(x_ref[...], w_ref[...], preferred_element_type=jnp.float32)
    h0 = h0 + b_ref[...]
    h0_ref[...] = h0
    g_ref[...] = h0 * dinv[:, None]
    dinv_ref[...] = dinv[:, None]


def _tc1(x, W1, b1, degp):
    grid = (N_PAD // _BR,)
    return pl.pallas_call(
        _tc1_body,
        grid=grid,
        in_specs=[
            pl.BlockSpec((_BR, F), lambda i: (i, 0)),
            pl.BlockSpec((F, F), lambda i: (0, 0)),
            pl.BlockSpec((1, F), lambda i: (0, 0)),
            pl.BlockSpec((NC, _BR), lambda i: (0, i)),
        ],
        out_specs=[
            pl.BlockSpec((_BR, F), lambda i: (i, 0)),
            pl.BlockSpec((_BR, F), lambda i: (i, 0)),
            pl.BlockSpec((_BR, 1), lambda i: (i, 0)),
        ],
        out_shape=[
            jax.ShapeDtypeStruct((N_PAD, F), jnp.float32),
            jax.ShapeDtypeStruct((N_PAD, F), jnp.float32),
            jax.ShapeDtypeStruct((N_PAD, 1), jnp.float32),
        ],
    )(x, W1, b1.reshape(1, F), degp)


def _combine_mid_body(a0_ref, a1_ref, g_ref, h0_ref, dinv_ref, gn_ref):
    dinv = dinv_ref[...]
    raw = a0_ref[...] + a1_ref[...] + g_ref[...]
    h = (1.0 - ALPHA) * dinv * raw + ALPHA * h0_ref[...]
    gn_ref[...] = h * dinv


def _combine_mid(aggs, g, h0, dinv):
    grid = (N_PAD // _BR,)
    bs = pl.BlockSpec((_BR, F), lambda i: (i, 0))
    return pl.pallas_call(
        _combine_mid_body,
        grid=grid,
        in_specs=[bs, bs, bs, bs, pl.BlockSpec((_BR, 1), lambda i: (i, 0))],
        out_specs=bs,
        out_shape=jax.ShapeDtypeStruct((N_PAD, F), jnp.float32),
    )(aggs[0], aggs[1], g, h0, dinv)


def _combine_lin_body(a0_ref, a1_ref, g_ref, h0_ref, dinv_ref, w_ref, b_ref,
                      h0n_ref, gn_ref):
    dinv = dinv_ref[...]
    raw = a0_ref[...] + a1_ref[...] + g_ref[...]
    h = (1.0 - ALPHA) * dinv * raw + ALPHA * h0_ref[...]
    h = jnp.maximum(h, 0.0)
    h2 = jnp.dot(h, w_ref[...], preferred_element_type=jnp.float32)
    h2 = h2 + b_ref[...]
    h0n_ref[...] = h2
    gn_ref[...] = h2 * dinv


def _combine_lin(aggs, g, h0, dinv, W2, b2):
    grid = (N_PAD // _BR,)
    bs = pl.BlockSpec((_BR, F), lambda i: (i, 0))
    return pl.pallas_call(
        _combine_lin_body,
        grid=grid,
        in_specs=[bs, bs, bs, bs,
                  pl.BlockSpec((_BR, 1), lambda i: (i, 0)),
                  pl.BlockSpec((F, F), lambda i: (0, 0)),
                  pl.BlockSpec((1, F), lambda i: (0, 0))],
        out_specs=[bs, bs],
        out_shape=[
            jax.ShapeDtypeStruct((N_PAD, F), jnp.float32),
            jax.ShapeDtypeStruct((N_PAD, F), jnp.float32),
        ],
    )(aggs[0], aggs[1], g, h0, dinv, W2, b2.reshape(1, F))


def _combine_final_body(a0_ref, a1_ref, g_ref, h0_ref, dinv_ref, out_ref):
    dinv = dinv_ref[...]
    raw = a0_ref[...] + a1_ref[...] + g_ref[...]
    out_ref[...] = (1.0 - ALPHA) * dinv * raw + ALPHA * h0_ref[...]


def _combine_final(aggs, g, h0, dinv):
    grid = (N_PAD // _BR,)
    bs = pl.BlockSpec((_BR, F), lambda i: (i, 0))
    return pl.pallas_call(
        _combine_final_body,
        grid=grid,
        in_specs=[bs, bs, bs, bs, pl.BlockSpec((_BR, 1), lambda i: (i, 0))],
        out_specs=bs,
        out_shape=jax.ShapeDtypeStruct((N_PAD, F), jnp.float32),
    )(aggs[0], aggs[1], g, h0, dinv)


def kernel(x, edge_index, W1, b1, W2, b2):
    x_pad = jnp.zeros((N_PAD, F), jnp.float32).at[:N_NODES].set(x)
    pad = jnp.full((E_PAD - E_EDGES,), N_NODES, jnp.int32)
    src = jnp.concatenate([edge_index[0], pad]).reshape(NC, NS, CPW, CHUNK)
    dst = jnp.concatenate([edge_index[1], pad]).reshape(NC, NS, CPW, CHUNK)

    degp = _degree_kernel(dst)
    h0, g, dinv = _tc1(x_pad, W1, b1, degp)

    out = None
    for k in range(K_STEPS):
        aggs = _scatter_kernel(g, src, dst)
        if k < K_STEPS - 1:
            g = _combine_mid(aggs, g, h0, dinv)
        else:
            h0, g = _combine_lin(aggs, g, h0, dinv, W2, b2)

    for k in range(K_STEPS):
        aggs = _scatter_kernel(g, src, dst)
        if k < K_STEPS - 1:
            g = _combine_mid(aggs, g, h0, dinv)
        else:
            out = _combine_final(aggs, g, h0, dinv)

    return out[:N_NODES]
